# Initial kernel scaffold; baseline (speedup 1.0000x reference)
import jax
import jax.numpy as jnp
from jax.experimental import pallas as pl


def kernel(l_feat, r_feat, P, Ws, als, ars, bs):
    raise NotImplementedError("write your pallas kernel here")


# speedup vs baseline: 12.2361x; 12.2361x over previous
import jax
import jax.numpy as jnp
from jax.experimental import pallas as pl

_H = 12
_DH = 64
_D = 768
_NLFT = 1568
_N = 3136
_RP = 2048
_PN = 196
_NEG = -3e38

_f32 = jnp.float32


def _mm(a, b):
    return jax.lax.dot_general(a, b, (((1,), (0,)), ((), ())),
                               preferred_element_type=jnp.float32)


def _leaky(x):
    return jnp.where(x >= 0, x, 0.2 * x)


def _elu(x):
    return jnp.where(x > 0, x, jnp.exp(x) - 1.0)


def _proj_lr_body(nl_ref, nr_ref, p_ref, zl_ref, zrt_ref):
    zl_ref[...] = _mm(nl_ref[...], p_ref[...])
    zr = _mm(nr_ref[...], p_ref[...])
    zrt_ref[...] = zr.T


def _topk_mask_body(zl_ref, zrt_ref, m_ref):
    s = _mm(zl_ref[...], zrt_ref[...])
    iota = jax.lax.broadcasted_iota(jnp.int32, s.shape, 1)
    msk = jnp.zeros(s.shape, _f32)
    for _ in range(4):
        mx = jnp.max(s, axis=1, keepdims=True)
        cand = jnp.where(s == mx, iota, jnp.int32(2**30))
        mn = jnp.min(cand, axis=1, keepdims=True)
        pick = cand == mn
        msk = jnp.where(pick, 1.0, msk)
        s = jnp.where(pick, _NEG, s)
    m_ref[...] = msk


def _make_proj(with_left, act):
    def body(x_ref, w_ref, al_ref, ar_ref, b_ref, *outs):
        z = _mm(x_ref[...], w_ref[...])
        outs[0][...] = z
        outs[1][...] = _mm(z, al_ref[...])
        outs[2][...] = _mm(z, ar_ref[...])
        if with_left:
            o = z[0:_NLFT, :] + b_ref[...]
            if act:
                o = _elu(o)
            outs[3][...] = o
    return body


def _proj(x, w, al, ar, b, with_left, act):
    shapes = [jax.ShapeDtypeStruct((_N, _D), _f32),
              jax.ShapeDtypeStruct((_N, _H), _f32),
              jax.ShapeDtypeStruct((_N, _H), _f32)]
    if with_left:
        shapes.append(jax.ShapeDtypeStruct((_NLFT, _D), _f32))
    return pl.pallas_call(_make_proj(with_left, act),
                          out_shape=tuple(shapes))(x, w, al, ar, b)


def _make_att1(act):
    def body(zl_ref, zrp_ref, ellt_ref, elr_ref, err_ref, mt_ref, b_ref,
             out_ref):
        mt = mt_ref[...]
        parts = []
        for h in range(_H):
            c0 = h * _DH
            er_h = err_ref[:, h:h + 1]
            els_h = elr_ref[:, h:h + 1]
            ell_h = ellt_ref[h:h + 1, :]
            em = jnp.where(mt > 0.5, _leaky(er_h + ell_h), _NEG)
            es = _leaky(els_h + er_h)
            mx = jnp.maximum(jnp.max(em, axis=1, keepdims=True), es)
            wt = jnp.exp(em - mx)
            ws = jnp.exp(es - mx)
            den = jnp.sum(wt, axis=1, keepdims=True) + ws
            num = _mm(wt, zl_ref[:, c0:c0 + _DH])
            parts.append((num + ws * zrp_ref[:, c0:c0 + _DH]) / den)
        o = jnp.concatenate(parts, axis=1) + b_ref[...]
        if act:
            o = _elu(o)
        out_ref[...] = o
    return body


def _att1(z_left, zr_pad, el_l_t, elr_pad, err_pad, mt, b, act):
    return pl.pallas_call(
        _make_att1(act),
        grid=(8,),
        in_specs=[
            pl.BlockSpec((_NLFT, _D), lambda p: (0, 0)),
            pl.BlockSpec((256, _D), lambda p: (p, 0)),
            pl.BlockSpec((_H, _NLFT), lambda p: (0, 0)),
            pl.BlockSpec((256, _H), lambda p: (p, 0)),
            pl.BlockSpec((256, _H), lambda p: (p, 0)),
            pl.BlockSpec((256, _NLFT), lambda p: (p, 0)),
            pl.BlockSpec((1, _D), lambda p: (0, 0)),
        ],
        out_specs=pl.BlockSpec((256, _D), lambda p: (p, 0)),
        out_shape=jax.ShapeDtypeStruct((_RP, _D), _f32),
    )(z_left, zr_pad, el_l_t, elr_pad, err_pad, mt, b)


def _make_att2(act):
    def body(x_ref, el_ref, er_ref, b_ref, s_ref, out_ref):
        z = x_ref[0]
        el = el_ref[0]
        er0 = er_ref[0][0:1, :]
        e = _leaky(el + er0)
        mx = jnp.max(e, axis=0, keepdims=True)
        w = jnp.exp(e - mx)
        w0 = w[0:1, :]
        den = jnp.sum(w, axis=0, keepdims=True) + w0
        sr = s_ref[...]
        wrep = _mm(w, sr)
        num = jnp.sum(wrep * z, axis=0, keepdims=True) + _mm(w0, sr) * z[0:1, :]
        head = num / _mm(den, sr)
        b = b_ref[...]
        rowi = jax.lax.broadcasted_iota(jnp.int32, (_PN, _D), 0)
        o = jnp.where(rowi == 0, head + b, z + b)
        if act:
            o = _elu(o)
        out_ref[0] = o
    return body


def _att2(x3, el3, er3, b, s12, act):
    return pl.pallas_call(
        _make_att2(act),
        grid=(16,),
        in_specs=[
            pl.BlockSpec((1, _PN, _D), lambda i: (i, 0, 0)),
            pl.BlockSpec((1, _PN, _H), lambda i: (i, 0, 0)),
            pl.BlockSpec((1, _PN, _H), lambda i: (i, 0, 0)),
            pl.BlockSpec((1, _D), lambda i: (0, 0)),
            pl.BlockSpec((_H, _D), lambda i: (0, 0)),
        ],
        out_specs=pl.BlockSpec((1, _PN, _D), lambda i: (i, 0, 0)),
        out_shape=jax.ShapeDtypeStruct((16, _PN, _D), _f32),
    )(x3, el3, er3, b, s12)


def kernel(l_feat, r_feat, P, Ws, als, ars, bs):
    node_l = l_feat.reshape(-1, _D).astype(_f32)
    node_r = r_feat.reshape(-1, _D).astype(_f32)

    zl, zr_t = pl.pallas_call(
        _proj_lr_body,
        out_shape=(jax.ShapeDtypeStruct((_NLFT, _D), _f32),
                   jax.ShapeDtypeStruct((_D, _NLFT), _f32)))(node_l, node_r, P)

    m = pl.pallas_call(
        _topk_mask_body,
        grid=(4,),
        in_specs=[pl.BlockSpec((392, _D), lambda i: (i, 0)),
                  pl.BlockSpec((_D, _NLFT), lambda i: (0, 0))],
        out_specs=pl.BlockSpec((392, _NLFT), lambda i: (i, 0)),
        out_shape=jax.ShapeDtypeStruct((_NLFT, _NLFT), _f32))(zl, zr_t)

    mt = jnp.pad(m.T, ((0, _RP - _NLFT), (0, 0)))

    eye = jnp.eye(_H, dtype=_f32)
    al_w = (als[:, :, :, None] * eye[:, None, :]).reshape(4, _D, _H)
    ar_w = (ars[:, :, :, None] * eye[:, None, :]).reshape(4, _D, _H)
    s12 = jnp.repeat(eye, _DH, axis=1)

    x = jnp.concatenate([node_l, node_r], axis=0)
    for layer in range(4):
        b = bs[layer].reshape(1, _D)
        act = layer < 3
        z, el, er, out_l = _proj(x, Ws[layer], al_w[layer], ar_w[layer], b,
                                 True, act)
        el_l_t = el[:_NLFT].T
        pad = ((0, _RP - _NLFT), (0, 0))
        out_r = _att1(z[:_NLFT], jnp.pad(z[_NLFT:], pad),
                      el_l_t, jnp.pad(el[_NLFT:], pad),
                      jnp.pad(er[_NLFT:], pad), mt, b, act)
        x = jnp.concatenate([out_l, out_r[:_NLFT]], axis=0)

    for layer in range(4):
        b = bs[layer].reshape(1, _D)
        act = layer < 3
        z, el, er = _proj(x, Ws[layer], al_w[layer], ar_w[layer], b,
                          False, act)
        out3 = _att2(z.reshape(16, _PN, _D), el.reshape(16, _PN, _H),
                     er.reshape(16, _PN, _H), b, s12, act)
        x = out3.reshape(_N, _D)

    return x.reshape(16, _PN, _D)[:, 0, :]


# speedup vs baseline: 15.8910x; 1.2987x over previous
import jax
import jax.numpy as jnp
from jax.experimental import pallas as pl

_H = 12
_DH = 64
_D = 768
_NLFT = 1568
_N = 3136
_RP = 2048
_PN = 196
_NEG = -3e38

_f32 = jnp.float32


def _mm(a, b):
    return jax.lax.dot_general(a, b, (((1,), (0,)), ((), ())),
                               preferred_element_type=jnp.float32)


def _leaky(x):
    return jnp.maximum(x, 0.2 * x)


def _elu(x):
    return jnp.where(x > 0, x, jnp.exp(x) - 1.0)


def _proj_lr_body(nl_ref, nr_ref, p_ref, zl_ref, zrt_ref):
    zl_ref[...] = _mm(nl_ref[...], p_ref[...])
    zr = _mm(nr_ref[...], p_ref[...])
    zrt_ref[...] = zr.T


def _topk_mask_body(zl_ref, zrt_ref, m_ref):
    s = _mm(zl_ref[...], zrt_ref[...])
    iota = jax.lax.broadcasted_iota(jnp.int32, s.shape, 1)
    msk = jnp.zeros(s.shape, _f32)
    for _ in range(4):
        mx = jnp.max(s, axis=1, keepdims=True)
        cand = jnp.where(s == mx, iota, jnp.int32(2**30))
        mn = jnp.min(cand, axis=1, keepdims=True)
        pick = cand == mn
        msk = jnp.where(pick, 1.0, msk)
        s = jnp.where(pick, _NEG, s)
    m_ref[...] = msk


def _make_proj(with_left, act):
    def body(x_ref, w_ref, al_ref, ar_ref, b_ref, *outs):
        z = _mm(x_ref[...], w_ref[...])
        outs[0][...] = z
        outs[1][...] = _mm(z, al_ref[...])
        outs[2][...] = _mm(z, ar_ref[...])
        if with_left:
            o = z[0:_NLFT, :] + b_ref[...]
            if act:
                o = _elu(o)
            outs[3][...] = o
    return body


def _proj(x, w, al, ar, b, with_left, act):
    shapes = [jax.ShapeDtypeStruct((_N, _D), _f32),
              jax.ShapeDtypeStruct((_N, _H), _f32),
              jax.ShapeDtypeStruct((_N, _H), _f32)]
    if with_left:
        shapes.append(jax.ShapeDtypeStruct((_NLFT, _D), _f32))
    return pl.pallas_call(_make_proj(with_left, act),
                          out_shape=tuple(shapes))(x, w, al, ar, b)


def _make_att1(act):
    def body(zl_ref, zr_ref, ellt_ref, elr_ref, err_ref, mt_ref, b_ref,
             out_ref):
        negadd = (mt_ref[...] - 1.0) * 3e38
        parts = []
        for h in range(_H):
            c0 = h * _DH
            er_h = err_ref[:, h:h + 1]
            els_h = elr_ref[:, h:h + 1]
            ell_h = ellt_ref[h:h + 1, 0:_NLFT]
            em = _leaky(er_h + ell_h) + negadd
            es = _leaky(els_h + er_h)
            mx = jnp.maximum(jnp.max(em, axis=1, keepdims=True), es)
            wt = jnp.exp(em - mx)
            ws = jnp.exp(es - mx)
            den = jnp.sum(wt, axis=1, keepdims=True) + ws
            num = _mm(wt, zl_ref[:, c0:c0 + _DH])
            parts.append((num + ws * zr_ref[:, c0:c0 + _DH]) / den)
        o = jnp.concatenate(parts, axis=1) + b_ref[...]
        if act:
            o = _elu(o)
        out_ref[...] = o
    return body


def _att1(z, el_t, el, er, mt, b, act):
    return pl.pallas_call(
        _make_att1(act),
        grid=(4,),
        in_specs=[
            pl.BlockSpec((_NLFT, _D), lambda p: (0, 0)),
            pl.BlockSpec((392, _D), lambda p: (p + 4, 0)),
            pl.BlockSpec((_H, _NLFT), lambda p: (0, 0)),
            pl.BlockSpec((392, _H), lambda p: (p + 4, 0)),
            pl.BlockSpec((392, _H), lambda p: (p + 4, 0)),
            pl.BlockSpec((392, _NLFT), lambda p: (p, 0)),
            pl.BlockSpec((1, _D), lambda p: (0, 0)),
        ],
        out_specs=pl.BlockSpec((392, _D), lambda p: (p, 0)),
        out_shape=jax.ShapeDtypeStruct((_NLFT, _D), _f32),
    )(z, z, el_t, el, er, mt, b)


def _make_att2(act):
    def body(x_ref, el_ref, er_ref, b_ref, s_ref, out_ref):
        z = x_ref[0]
        el = el_ref[0]
        er0 = er_ref[0][0:1, :]
        e = _leaky(el + er0)
        mx = jnp.max(e, axis=0, keepdims=True)
        w = jnp.exp(e - mx)
        w0 = w[0:1, :]
        den = jnp.sum(w, axis=0, keepdims=True) + w0
        sr = s_ref[...]
        wrep = _mm(w, sr)
        num = jnp.sum(wrep * z, axis=0, keepdims=True) + _mm(w0, sr) * z[0:1, :]
        head = num / _mm(den, sr)
        b = b_ref[...]
        rowi = jax.lax.broadcasted_iota(jnp.int32, (_PN, _D), 0)
        o = jnp.where(rowi == 0, head + b, z + b)
        if act:
            o = _elu(o)
        out_ref[0] = o
    return body


def _att2(x3, el3, er3, b, s12, act):
    return pl.pallas_call(
        _make_att2(act),
        grid=(16,),
        in_specs=[
            pl.BlockSpec((1, _PN, _D), lambda i: (i, 0, 0)),
            pl.BlockSpec((1, _PN, _H), lambda i: (i, 0, 0)),
            pl.BlockSpec((1, _PN, _H), lambda i: (i, 0, 0)),
            pl.BlockSpec((1, _D), lambda i: (0, 0)),
            pl.BlockSpec((_H, _D), lambda i: (0, 0)),
        ],
        out_specs=pl.BlockSpec((1, _PN, _D), lambda i: (i, 0, 0)),
        out_shape=jax.ShapeDtypeStruct((16, _PN, _D), _f32),
    )(x3, el3, er3, b, s12)


def kernel(l_feat, r_feat, P, Ws, als, ars, bs):
    node_l = l_feat.reshape(-1, _D).astype(_f32)
    node_r = r_feat.reshape(-1, _D).astype(_f32)

    zl, zr_t = pl.pallas_call(
        _proj_lr_body,
        out_shape=(jax.ShapeDtypeStruct((_NLFT, _D), _f32),
                   jax.ShapeDtypeStruct((_D, _NLFT), _f32)))(node_l, node_r, P)

    m = pl.pallas_call(
        _topk_mask_body,
        grid=(4,),
        in_specs=[pl.BlockSpec((392, _D), lambda i: (i, 0)),
                  pl.BlockSpec((_D, _NLFT), lambda i: (0, 0))],
        out_specs=pl.BlockSpec((392, _NLFT), lambda i: (i, 0)),
        out_shape=jax.ShapeDtypeStruct((_NLFT, _NLFT), _f32))(zl, zr_t)

    mt = m.T

    eye = jnp.eye(_H, dtype=_f32)
    al_w = (als[:, :, :, None] * eye[:, None, :]).reshape(4, _D, _H)
    ar_w = (ars[:, :, :, None] * eye[:, None, :]).reshape(4, _D, _H)
    s12 = jnp.repeat(eye, _DH, axis=1)

    x = jnp.concatenate([node_l, node_r], axis=0)
    for layer in range(4):
        b = bs[layer].reshape(1, _D)
        act = layer < 3
        z, el, er, out_l = _proj(x, Ws[layer], al_w[layer], ar_w[layer], b,
                                 True, act)
        out_r = _att1(z, el[:_NLFT].T, el, er, mt, b, act)
        x = jnp.concatenate([out_l, out_r], axis=0)

    for layer in range(4):
        b = bs[layer].reshape(1, _D)
        act = layer < 3
        z, el, er = _proj(x, Ws[layer], al_w[layer], ar_w[layer], b,
                          False, act)
        out3 = _att2(z.reshape(16, _PN, _D), el.reshape(16, _PN, _H),
                     er.reshape(16, _PN, _H), b, s12, act)
        x = out3.reshape(_N, _D)

    return x.reshape(16, _PN, _D)[:, 0, :]


# speedup vs baseline: 19.1285x; 1.2037x over previous
import jax
import jax.numpy as jnp
from jax.experimental import pallas as pl

_H = 12
_DH = 64
_D = 768
_NLFT = 1568
_N = 3136
_RP = 2048
_PN = 196
_NEG = -3e38

_f32 = jnp.float32


def _mm(a, b):
    return jax.lax.dot_general(a, b, (((1,), (0,)), ((), ())),
                               preferred_element_type=jnp.float32)


def _leaky(x):
    return jnp.maximum(x, 0.2 * x)


def _elu(x):
    return jnp.where(x > 0, x, jnp.exp(x) - 1.0)


def _proj_lr_body(nl_ref, nr_ref, p_ref, zl_ref, zrt_ref):
    zl_ref[...] = _mm(nl_ref[...], p_ref[...])
    zr = _mm(nr_ref[...], p_ref[...])
    zrt_ref[...] = zr.T


def _topk_mask_body(zl_ref, zrt_ref, m_ref):
    s = _mm(zl_ref[...], zrt_ref[...])
    iota = jax.lax.broadcasted_iota(jnp.int32, s.shape, 1)
    msk = jnp.zeros(s.shape, _f32)
    for _ in range(4):
        mx = jnp.max(s, axis=1, keepdims=True)
        cand = jnp.where(s == mx, iota, jnp.int32(2**30))
        mn = jnp.min(cand, axis=1, keepdims=True)
        pick = cand == mn
        msk = jnp.where(pick, 1.0, msk)
        s = jnp.where(pick, _NEG, s)
    m_ref[...] = msk


def _make_proj(with_left, act):
    def body(x_ref, w_ref, al_ref, ar_ref, b_ref, *outs):
        z = _mm(x_ref[...], w_ref[...])
        outs[0][...] = z
        outs[1][...] = _mm(z, al_ref[...])
        outs[2][...] = _mm(z, ar_ref[...])
        if with_left:
            o = z[0:_NLFT, :] + b_ref[...]
            if act:
                o = _elu(o)
            outs[3][...] = o
    return body


def _proj(x, w, al, ar, b, with_left, act):
    shapes = [jax.ShapeDtypeStruct((_N, _D), _f32),
              jax.ShapeDtypeStruct((_N, _H), _f32),
              jax.ShapeDtypeStruct((_N, _H), _f32)]
    if with_left:
        shapes.append(jax.ShapeDtypeStruct((_NLFT, _D), _f32))
    return pl.pallas_call(_make_proj(with_left, act),
                          out_shape=tuple(shapes))(x, w, al, ar, b)


def _make_att1(act):
    def body(zl_ref, zr_ref, ellt_ref, elr_ref, err_ref, mt_ref, b_ref,
             out_ref):
        negadd = (mt_ref[...] - 1.0) * 3e38
        parts = []
        for h in range(_H):
            c0 = h * _DH
            er_h = err_ref[:, h:h + 1]
            els_h = elr_ref[:, h:h + 1]
            ell_h = ellt_ref[h:h + 1, 0:_NLFT]
            em = _leaky(er_h + ell_h) + negadd
            es = _leaky(els_h + er_h)
            mx = jnp.maximum(jnp.max(em, axis=1, keepdims=True), es)
            wt = jnp.exp(em - mx)
            ws = jnp.exp(es - mx)
            den = jnp.sum(wt, axis=1, keepdims=True) + ws
            num = _mm(wt, zl_ref[:, c0:c0 + _DH])
            parts.append((num + ws * zr_ref[:, c0:c0 + _DH]) * (1.0 / den))
        o = jnp.concatenate(parts, axis=1) + b_ref[...]
        if act:
            o = _elu(o)
        out_ref[...] = o
    return body


def _att1(z, el_t, el, er, mt, b, act):
    return pl.pallas_call(
        _make_att1(act),
        grid=(4,),
        in_specs=[
            pl.BlockSpec((_NLFT, _D), lambda p: (0, 0)),
            pl.BlockSpec((392, _D), lambda p: (p + 4, 0)),
            pl.BlockSpec((_H, _NLFT), lambda p: (0, 0)),
            pl.BlockSpec((392, _H), lambda p: (p + 4, 0)),
            pl.BlockSpec((392, _H), lambda p: (p + 4, 0)),
            pl.BlockSpec((392, _NLFT), lambda p: (p, 0)),
            pl.BlockSpec((1, _D), lambda p: (0, 0)),
        ],
        out_specs=pl.BlockSpec((392, _D), lambda p: (p, 0)),
        out_shape=jax.ShapeDtypeStruct((_NLFT, _D), _f32),
    )(z, z, el_t, el, er, mt, b)


def _make_att2(act):
    def body(x_ref, w_ref, al_ref, ar_ref, b_ref, s_ref, out_ref):
        z = _mm(x_ref[0], w_ref[...])
        el = _mm(z, al_ref[...])
        er0 = _mm(z[0:1, :], ar_ref[...])
        e = _leaky(el + er0)
        mx = jnp.max(e, axis=0, keepdims=True)
        w = jnp.exp(e - mx)
        w0 = w[0:1, :]
        den = jnp.sum(w, axis=0, keepdims=True) + w0
        sr = s_ref[...]
        wrep = _mm(w, sr)
        num = jnp.sum(wrep * z, axis=0, keepdims=True) + _mm(w0, sr) * z[0:1, :]
        head = num / _mm(den, sr)
        b = b_ref[...]
        rowi = jax.lax.broadcasted_iota(jnp.int32, (_PN, _D), 0)
        o = jnp.where(rowi == 0, head + b, z + b)
        if act:
            o = _elu(o)
        out_ref[0] = o
    return body


def _att2(x3, w, al, ar, b, s12, act):
    return pl.pallas_call(
        _make_att2(act),
        grid=(16,),
        in_specs=[
            pl.BlockSpec((1, _PN, _D), lambda i: (i, 0, 0)),
            pl.BlockSpec((_D, _D), lambda i: (0, 0)),
            pl.BlockSpec((_D, _H), lambda i: (0, 0)),
            pl.BlockSpec((_D, _H), lambda i: (0, 0)),
            pl.BlockSpec((1, _D), lambda i: (0, 0)),
            pl.BlockSpec((_H, _D), lambda i: (0, 0)),
        ],
        out_specs=pl.BlockSpec((1, _PN, _D), lambda i: (i, 0, 0)),
        out_shape=jax.ShapeDtypeStruct((16, _PN, _D), _f32),
    )(x3, w, al, ar, b, s12)


def kernel(l_feat, r_feat, P, Ws, als, ars, bs):
    node_l = l_feat.reshape(-1, _D).astype(_f32)
    node_r = r_feat.reshape(-1, _D).astype(_f32)

    zl, zr_t = pl.pallas_call(
        _proj_lr_body,
        out_shape=(jax.ShapeDtypeStruct((_NLFT, _D), _f32),
                   jax.ShapeDtypeStruct((_D, _NLFT), _f32)))(node_l, node_r, P)

    m = pl.pallas_call(
        _topk_mask_body,
        grid=(4,),
        in_specs=[pl.BlockSpec((392, _D), lambda i: (i, 0)),
                  pl.BlockSpec((_D, _NLFT), lambda i: (0, 0))],
        out_specs=pl.BlockSpec((392, _NLFT), lambda i: (i, 0)),
        out_shape=jax.ShapeDtypeStruct((_NLFT, _NLFT), _f32))(zl, zr_t)

    mt = m.T

    eye = jnp.eye(_H, dtype=_f32)
    al_w = (als[:, :, :, None] * eye[:, None, :]).reshape(4, _D, _H)
    ar_w = (ars[:, :, :, None] * eye[:, None, :]).reshape(4, _D, _H)
    s12 = jnp.repeat(eye, _DH, axis=1)

    x = jnp.concatenate([node_l, node_r], axis=0)
    for layer in range(4):
        b = bs[layer].reshape(1, _D)
        act = layer < 3
        z, el, er, out_l = _proj(x, Ws[layer], al_w[layer], ar_w[layer], b,
                                 True, act)
        out_r = _att1(z, el[:_NLFT].T, el, er, mt, b, act)
        x = jnp.concatenate([out_l, out_r], axis=0)

    x3 = x.reshape(16, _PN, _D)
    for layer in range(4):
        b = bs[layer].reshape(1, _D)
        act = layer < 3
        x3 = _att2(x3, Ws[layer], al_w[layer], ar_w[layer], b, s12, act)

    return x3[:, 0, :]


# speedup vs baseline: 20.1292x; 1.0523x over previous
import jax
import jax.numpy as jnp
from jax.experimental import pallas as pl

_H = 12
_DH = 64
_D = 768
_NLFT = 1568
_N = 3136
_RP = 2048
_PN = 196
_NEG = -3e38

_f32 = jnp.float32


def _mm(a, b):
    return jax.lax.dot_general(a, b, (((1,), (0,)), ((), ())),
                               preferred_element_type=jnp.float32)


def _leaky(x):
    return jnp.maximum(x, 0.2 * x)


def _elu(x):
    return jnp.where(x > 0, x, jnp.exp(x) - 1.0)


def _proj_lr_body(nl_ref, nr_ref, p_ref, zl_ref, zrt_ref):
    zl_ref[...] = _mm(nl_ref[...], p_ref[...])
    zr = _mm(nr_ref[...], p_ref[...])
    zrt_ref[...] = zr.T


def _topk_mask_body(zl_ref, zrt_ref, m_ref):
    s = _mm(zl_ref[...], zrt_ref[...])
    iota = jax.lax.broadcasted_iota(jnp.int32, s.shape, 1)
    msk = jnp.zeros(s.shape, _f32)
    for _ in range(4):
        mx = jnp.max(s, axis=1, keepdims=True)
        cand = jnp.where(s == mx, iota, jnp.int32(2**30))
        mn = jnp.min(cand, axis=1, keepdims=True)
        pick = cand == mn
        msk = jnp.where(pick, 1.0, msk)
        s = jnp.where(pick, _NEG, s)
    m_ref[...] = msk


def _make_proj(with_left, act):
    def body(x_ref, w_ref, al_ref, ar_ref, b_ref, *outs):
        z = _mm(x_ref[...], w_ref[...])
        outs[0][...] = z
        outs[1][...] = _mm(z, al_ref[...])
        outs[2][...] = _mm(z, ar_ref[...])
        if with_left:
            o = z[0:_NLFT, :] + b_ref[...]
            if act:
                o = _elu(o)
            outs[3][...] = o
    return body


def _proj(x, w, al, ar, b, with_left, act):
    shapes = [jax.ShapeDtypeStruct((_N, _D), _f32),
              jax.ShapeDtypeStruct((_N, _H), _f32),
              jax.ShapeDtypeStruct((_N, _H), _f32)]
    if with_left:
        shapes.append(jax.ShapeDtypeStruct((_NLFT, _D), _f32))
    return pl.pallas_call(_make_proj(with_left, act),
                          out_shape=tuple(shapes))(x, w, al, ar, b)


def _make_att1(act):
    def body(zl_ref, zr_ref, ellt_ref, elr_ref, err_ref, mt_ref, b_ref,
             out_ref):
        negadd = (mt_ref[...] - 1.0) * 3e38
        parts = []
        for h in range(_H):
            c0 = h * _DH
            er_h = err_ref[:, h:h + 1]
            els_h = elr_ref[:, h:h + 1]
            ell_h = ellt_ref[h:h + 1, 0:_NLFT]
            em = _leaky(er_h + ell_h) + negadd
            es = _leaky(els_h + er_h)
            mx = jnp.maximum(jnp.max(em, axis=1, keepdims=True), es)
            wt = jnp.exp(em - mx)
            ws = jnp.exp(es - mx)
            den = jnp.sum(wt, axis=1, keepdims=True) + ws
            num = _mm(wt, zl_ref[:, c0:c0 + _DH])
            parts.append((num + ws * zr_ref[:, c0:c0 + _DH]) * (1.0 / den))
        o = jnp.concatenate(parts, axis=1) + b_ref[...]
        if act:
            o = _elu(o)
        out_ref[...] = o
    return body


def _att1(z, el_t, el, er, mt, b, act):
    return pl.pallas_call(
        _make_att1(act),
        grid=(4,),
        in_specs=[
            pl.BlockSpec((_NLFT, _D), lambda p: (0, 0)),
            pl.BlockSpec((392, _D), lambda p: (p + 4, 0)),
            pl.BlockSpec((_H, _NLFT), lambda p: (0, 0)),
            pl.BlockSpec((392, _H), lambda p: (p + 4, 0)),
            pl.BlockSpec((392, _H), lambda p: (p + 4, 0)),
            pl.BlockSpec((392, _NLFT), lambda p: (p, 0)),
            pl.BlockSpec((1, _D), lambda p: (0, 0)),
        ],
        out_specs=pl.BlockSpec((392, _D), lambda p: (p, 0)),
        out_shape=jax.ShapeDtypeStruct((_NLFT, _D), _f32),
    )(z, z, el_t, el, er, mt, b)


_IPB = 4


def _make_att2(act):
    def body(x_ref, w_ref, al_ref, ar_ref, b_ref, s_ref, out_ref):
        zb = _mm(x_ref[...].reshape(_IPB * _PN, _D), w_ref[...])
        elb = _mm(zb, al_ref[...])
        erb = _mm(zb, ar_ref[...])
        sr = s_ref[...]
        b = b_ref[...]
        rowi = jax.lax.broadcasted_iota(jnp.int32, (_PN, _D), 0)
        for k in range(_IPB):
            r0 = k * _PN
            z = zb[r0:r0 + _PN]
            el = elb[r0:r0 + _PN]
            er0 = erb[r0:r0 + 1]
            e = _leaky(el + er0)
            mx = jnp.max(e, axis=0, keepdims=True)
            w = jnp.exp(e - mx)
            w0 = w[0:1, :]
            den = jnp.sum(w, axis=0, keepdims=True) + w0
            wrep = _mm(w, sr)
            num = (jnp.sum(wrep * z, axis=0, keepdims=True)
                   + _mm(w0, sr) * z[0:1, :])
            head = num / _mm(den, sr)
            o = jnp.where(rowi == 0, head + b, z + b)
            if act:
                o = _elu(o)
            out_ref[k] = o
    return body


def _att2(x3, w, al, ar, b, s12, act):
    return pl.pallas_call(
        _make_att2(act),
        grid=(16 // _IPB,),
        in_specs=[
            pl.BlockSpec((_IPB, _PN, _D), lambda i: (i, 0, 0)),
            pl.BlockSpec((_D, _D), lambda i: (0, 0)),
            pl.BlockSpec((_D, _H), lambda i: (0, 0)),
            pl.BlockSpec((_D, _H), lambda i: (0, 0)),
            pl.BlockSpec((1, _D), lambda i: (0, 0)),
            pl.BlockSpec((_H, _D), lambda i: (0, 0)),
        ],
        out_specs=pl.BlockSpec((_IPB, _PN, _D), lambda i: (i, 0, 0)),
        out_shape=jax.ShapeDtypeStruct((16, _PN, _D), _f32),
    )(x3, w, al, ar, b, s12)


def kernel(l_feat, r_feat, P, Ws, als, ars, bs):
    node_l = l_feat.reshape(-1, _D).astype(_f32)
    node_r = r_feat.reshape(-1, _D).astype(_f32)

    zl, zr_t = pl.pallas_call(
        _proj_lr_body,
        out_shape=(jax.ShapeDtypeStruct((_NLFT, _D), _f32),
                   jax.ShapeDtypeStruct((_D, _NLFT), _f32)))(node_l, node_r, P)

    m = pl.pallas_call(
        _topk_mask_body,
        grid=(4,),
        in_specs=[pl.BlockSpec((392, _D), lambda i: (i, 0)),
                  pl.BlockSpec((_D, _NLFT), lambda i: (0, 0))],
        out_specs=pl.BlockSpec((392, _NLFT), lambda i: (i, 0)),
        out_shape=jax.ShapeDtypeStruct((_NLFT, _NLFT), _f32))(zl, zr_t)

    mt = m.T

    eye = jnp.eye(_H, dtype=_f32)
    al_w = (als[:, :, :, None] * eye[:, None, :]).reshape(4, _D, _H)
    ar_w = (ars[:, :, :, None] * eye[:, None, :]).reshape(4, _D, _H)
    s12 = jnp.repeat(eye, _DH, axis=1)

    x = jnp.concatenate([node_l, node_r], axis=0)
    for layer in range(4):
        b = bs[layer].reshape(1, _D)
        act = layer < 3
        z, el, er, out_l = _proj(x, Ws[layer], al_w[layer], ar_w[layer], b,
                                 True, act)
        out_r = _att1(z, el[:_NLFT].T, el, er, mt, b, act)
        x = jnp.concatenate([out_l, out_r], axis=0)

    x3 = x.reshape(16, _PN, _D)
    for layer in range(4):
        b = bs[layer].reshape(1, _D)
        act = layer < 3
        x3 = _att2(x3, Ws[layer], al_w[layer], ar_w[layer], b, s12, act)

    return x3[:, 0, :]
